# baseline (speedup 1.0000x reference)
import functools

import jax
import jax.numpy as jnp
from jax import lax
from jax.experimental import pallas as pl
from jax.experimental.pallas import tpu as pltpu
from jax.experimental.pallas import tpu_sc as plsc

N = 10000
D = 128
K = 64
E = 320000
TEMP = 0.5
LAMDA = 0.01

CHUNK = 80
NCORES = 2
NSUB = 16
NWORK = NCORES * NSUB
NCH = (E // NWORK) // CHUNK
ZROWS = N // NSUB
SROWS = N // NSUB
DEGW = 16
NBUF = 5

RB = 1000
GRID = N // RB

_HI = lax.Precision.HIGHEST


def _dot(a, b, dims, precision=None):
    return lax.dot_general(a, b, (dims, ((), ())),
                           preferred_element_type=jnp.float32,
                           precision=precision)


_sc_mesh = plsc.VectorSubcoreMesh(core_axis_name="c", subcore_axis_name="s")


@functools.partial(
    pl.kernel,
    out_type=[jax.ShapeDtypeStruct((NCORES, N, K), jnp.float32),
              jax.ShapeDtypeStruct((NCORES, N, DEGW), jnp.float32)],
    mesh=_sc_mesh,
    compiler_params=pltpu.CompilerParams(use_tc_tiling_on_sc=False),
    scratch_types=[
        pltpu.VMEM((NCH, CHUNK), jnp.int32),
        pltpu.VMEM((NCH, CHUNK), jnp.int32),
        [pltpu.VMEM((CHUNK, K), jnp.float32) for _ in range(NBUF)],
        pltpu.VMEM((CHUNK, DEGW), jnp.float32),
        pltpu.VMEM_SHARED((N, K), jnp.float32),
        pltpu.VMEM_SHARED((N, DEGW), jnp.float32),
        [pltpu.SemaphoreType.DMA for _ in range(NBUF)],
        [pltpu.SemaphoreType.DMA for _ in range(NBUF)],
        pltpu.SemaphoreType.DMA,
    ],
)
def _sc_apply_deg(hp, ei, zf, zd, ones, outf, outd,
                  srcv, dstv, rows, onesv, featS, degS, gsem, ssem, dsem):
    c = lax.axis_index("c")
    s = lax.axis_index("s")
    wid = c * NSUB + s
    row0 = s * SROWS
    pltpu.sync_copy(zf, featS.at[pl.ds(s * ZROWS, ZROWS)])
    pltpu.sync_copy(zd, degS.at[pl.ds(s * ZROWS, ZROWS)])
    pltpu.sync_copy(ei.at[0, wid], srcv)
    pltpu.sync_copy(ei.at[1, wid], dstv)
    pltpu.sync_copy(ones, onesv)
    plsc.subcore_barrier()

    for b in range(NBUF):
        pltpu.async_copy(hp.at[srcv.at[b]], rows[b], gsem[b])

    def step(g, carry):
        j0 = NBUF * g
        for b in range(NBUF):
            pltpu.make_async_copy(hp.at[srcv.at[j0 + b]], rows[b], gsem[b]).wait()
            pltpu.async_copy(rows[b], featS.at[dstv.at[j0 + b]], ssem[b], add=True)
            pltpu.async_copy(onesv, degS.at[dstv.at[j0 + b]], dsem, add=True)
        for b in range(NBUF):
            pltpu.make_async_copy(rows[b], featS.at[dstv.at[j0 + b]], ssem[b]).wait()

            @pl.when(j0 + NBUF + b < NCH)
            def _():
                pltpu.async_copy(hp.at[srcv.at[j0 + NBUF + b]], rows[b], gsem[b])
        return carry

    lax.fori_loop(0, NCH // NBUF, step, 0)
    pltpu.make_async_copy(outd.at[c], degS, dsem).wait()
    plsc.subcore_barrier()
    pltpu.sync_copy(featS.at[pl.ds(row0, SROWS)],
                    outf.at[c, pl.ds(row0, SROWS)])
    pltpu.sync_copy(degS.at[pl.ds(row0, SROWS)],
                    outd.at[c, pl.ds(row0, SROWS)])


@functools.partial(
    pl.kernel,
    out_type=[jax.ShapeDtypeStruct((NCORES, N, K), jnp.float32)],
    mesh=_sc_mesh,
    compiler_params=pltpu.CompilerParams(use_tc_tiling_on_sc=False),
    scratch_types=[
        pltpu.VMEM((NCH, CHUNK), jnp.int32),
        pltpu.VMEM((NCH, CHUNK), jnp.int32),
        [pltpu.VMEM((CHUNK, K), jnp.float32) for _ in range(NBUF)],
        pltpu.VMEM_SHARED((N, K), jnp.float32),
        [pltpu.SemaphoreType.DMA for _ in range(NBUF)],
        [pltpu.SemaphoreType.DMA for _ in range(NBUF)],
    ],
)
def _sc_apply(hp, ei, zf, outf, srcv, dstv, rows, featS, gsem, ssem):
    c = lax.axis_index("c")
    s = lax.axis_index("s")
    wid = c * NSUB + s
    row0 = s * SROWS
    pltpu.sync_copy(zf, featS.at[pl.ds(s * ZROWS, ZROWS)])
    pltpu.sync_copy(ei.at[0, wid], srcv)
    pltpu.sync_copy(ei.at[1, wid], dstv)
    plsc.subcore_barrier()

    for b in range(NBUF):
        pltpu.async_copy(hp.at[srcv.at[b]], rows[b], gsem[b])

    def step(g, carry):
        j0 = NBUF * g
        for b in range(NBUF):
            pltpu.make_async_copy(hp.at[srcv.at[j0 + b]], rows[b], gsem[b]).wait()
            pltpu.async_copy(rows[b], featS.at[dstv.at[j0 + b]], ssem[b], add=True)
        for b in range(NBUF):
            pltpu.make_async_copy(rows[b], featS.at[dstv.at[j0 + b]], ssem[b]).wait()

            @pl.when(j0 + NBUF + b < NCH)
            def _():
                pltpu.async_copy(hp.at[srcv.at[j0 + NBUF + b]], rows[b], gsem[b])
        return carry

    lax.fori_loop(0, NCH // NBUF, step, 0)
    plsc.subcore_barrier()
    pltpu.sync_copy(featS.at[pl.ds(row0, SROWS)],
                    outf.at[c, pl.ds(row0, SROWS)])


def _mm1_body(x_ref, w_ref, b_ref, o_ref):
    o_ref[...] = _dot(x_ref[...], w_ref[...], ((1,), (0,))) + b_ref[...]


_mm1 = pl.pallas_call(
    _mm1_body,
    grid=(GRID,),
    in_specs=[pl.BlockSpec((RB, D), lambda i: (i, 0)),
              pl.BlockSpec((D, K), lambda i: (0, 0)),
              pl.BlockSpec((1, K), lambda i: (0, 0))],
    out_specs=pl.BlockSpec((RB, K), lambda i: (i, 0)),
    out_shape=jax.ShapeDtypeStruct((N, K), jnp.float32),
)


def _mid_body(f0_ref, f1_ref, d0_ref, d1_ref, w_ref, b_ref, o_ref):
    deg = jnp.maximum(d0_ref[0][:, 0:1] + d1_ref[0][:, 0:1], 1.0)
    h = jnp.maximum((f0_ref[0] + f1_ref[0]) / deg, 0.0)
    o_ref[...] = _dot(h, w_ref[...], ((1,), (0,))) + b_ref[...]


_mid = pl.pallas_call(
    _mid_body,
    grid=(GRID,),
    in_specs=[pl.BlockSpec((1, RB, K), lambda i: (0, i, 0)),
              pl.BlockSpec((1, RB, K), lambda i: (1, i, 0)),
              pl.BlockSpec((1, RB, DEGW), lambda i: (0, i, 0)),
              pl.BlockSpec((1, RB, DEGW), lambda i: (1, i, 0)),
              pl.BlockSpec((K, K), lambda i: (0, 0)),
              pl.BlockSpec((1, K), lambda i: (0, 0))],
    out_specs=pl.BlockSpec((RB, K), lambda i: (i, 0)),
    out_shape=jax.ShapeDtypeStruct((N, K), jnp.float32),
)


def _final_body(f0_ref, f1_ref, d0_ref, d1_ref, emb_ref, eye_ref,
                gamma_ref, cidx_ref, loss_ref, miu_ref,
                gbuf, gsum_s, miun_s, tsum_s, tmiun_s):
    i = pl.program_id(0)

    @pl.when(i < GRID)
    def _():
        deg = jnp.maximum(d0_ref[0][:, 0:1] + d1_ref[0][:, 0:1], 1.0)
        x = (f0_ref[0] + f1_ref[0]) / deg
        m = jnp.max(x, axis=-1, keepdims=True)
        e = jnp.exp(x - m)
        gamma = e / jnp.sum(e, axis=-1, keepdims=True)
        gamma_ref[...] = gamma
        gbuf[pl.ds(i * RB, RB), :] = gamma
        cols = lax.broadcasted_iota(jnp.int32, (RB, K), 1)
        cidx_ref[...] = jnp.min(jnp.where(x >= m, cols, K), axis=-1,
                                keepdims=True)
        gs = jnp.sum(gamma, axis=0, keepdims=True)
        mn = _dot(gamma, emb_ref[...], ((0,), (0,)))

        @pl.when(i == 0)
        def _():
            gsum_s[...] = gs
            miun_s[...] = mn

        @pl.when(i > 0)
        def _():
            gsum_s[...] += gs
            miun_s[...] += mn

    @pl.when((i >= GRID) & (i < 2 * GRID))
    def _():
        g = gbuf[pl.ds((i - GRID) * RB, RB), :]
        y = g * g / gsum_s[...]
        m = jnp.max(y, axis=-1, keepdims=True)
        e = jnp.exp(y - m)
        tg = e / jnp.sum(e, axis=-1, keepdims=True)
        ts = jnp.sum(tg, axis=0, keepdims=True)
        tm = _dot(tg, emb_ref[...], ((0,), (0,)))

        @pl.when(i == GRID)
        def _():
            tsum_s[...] = ts
            tmiun_s[...] = tm

        @pl.when(i > GRID)
        def _():
            tsum_s[...] += ts
            tmiun_s[...] += tm

    @pl.when(i == 2 * GRID)
    def _():
        gs = gsum_s[...]
        eye = eye_ref[...]
        gcol = _dot(eye, gs, ((1,), (1,)), precision=_HI)
        tcol = _dot(eye, tsum_s[...], ((1,), (1,)), precision=_HI)
        miu = miun_s[...] / gcol
        tmiu = tmiun_s[...] / tcol
        miu_ref[...] = miu
        na = jnp.maximum(jnp.sqrt(jnp.sum(miu * miu, axis=-1, keepdims=True)),
                         1e-8)
        nb = jnp.maximum(jnp.sqrt(jnp.sum(tmiu * tmiu, axis=-1, keepdims=True)),
                         1e-8)
        a = miu / na
        b = tmiu / nb
        sim = _dot(a, b, ((1,), (1,))) / TEMP
        rm = jnp.max(sim, axis=-1, keepdims=True)
        lse_r = jnp.log(jnp.sum(jnp.exp(sim - rm), axis=-1, keepdims=True)) + rm
        cm = jnp.max(sim, axis=0, keepdims=True)
        lse_c = jnp.log(jnp.sum(jnp.exp(sim - cm), axis=0, keepdims=True)) + cm
        diag_ab = jnp.sum((sim - lse_r) * eye) / K
        diag_ba = jnp.sum((sim - lse_c) * eye) / K
        cl = -0.5 * (diag_ab + diag_ba)
        reg = jnp.mean(gs * gs) * LAMDA
        loss_ref[...] = jnp.broadcast_to(cl + reg, (1, 1))


def _cap(i):
    return jnp.minimum(i, GRID - 1)


_final = pl.pallas_call(
    _final_body,
    grid=(2 * GRID + 1,),
    in_specs=[pl.BlockSpec((1, RB, K), lambda i: (0, _cap(i), 0)),
              pl.BlockSpec((1, RB, K), lambda i: (1, _cap(i), 0)),
              pl.BlockSpec((1, RB, DEGW), lambda i: (0, _cap(i), 0)),
              pl.BlockSpec((1, RB, DEGW), lambda i: (1, _cap(i), 0)),
              pl.BlockSpec((RB, D),
                           lambda i: (jnp.where(i < GRID, i,
                                                _cap(i - GRID)), 0)),
              pl.BlockSpec((K, K), lambda i: (0, 0))],
    out_specs=[pl.BlockSpec((RB, K), lambda i: (_cap(i), 0)),
               pl.BlockSpec((RB, 1), lambda i: (_cap(i), 0)),
               pl.BlockSpec((1, 1), lambda i: (0, 0)),
               pl.BlockSpec((K, D), lambda i: (0, 0))],
    out_shape=[jax.ShapeDtypeStruct((N, K), jnp.float32),
               jax.ShapeDtypeStruct((N, 1), jnp.int32),
               jax.ShapeDtypeStruct((1, 1), jnp.float32),
               jax.ShapeDtypeStruct((K, D), jnp.float32)],
    scratch_shapes=[pltpu.VMEM((N, K), jnp.float32),
                    pltpu.VMEM((1, K), jnp.float32),
                    pltpu.VMEM((K, D), jnp.float32),
                    pltpu.VMEM((1, K), jnp.float32),
                    pltpu.VMEM((K, D), jnp.float32)],
)


def kernel(embeds, edge_index, W0, b0, W1, b1):
    ei = edge_index.reshape(2, NWORK, NCH, CHUNK)
    zf = jnp.zeros((ZROWS, K), jnp.float32)
    zd = jnp.zeros((ZROWS, DEGW), jnp.float32)
    ones = jnp.ones((CHUNK, DEGW), jnp.float32)
    eye = jnp.eye(K, dtype=jnp.float32)

    hp1 = _mm1(embeds, W0, b0.reshape(1, K))
    aggf1, aggd = _sc_apply_deg(hp1, ei, zf, zd, ones)
    hp2 = _mid(aggf1, aggf1, aggd, aggd, W1, b1.reshape(1, K))
    (aggf2,) = _sc_apply(hp2, ei, zf)
    gamma, cidx, loss11, miu = _final(aggf2, aggf2, aggd, aggd, embeds, eye)
    return (loss11[0, 0], gamma, cidx.reshape(N), miu)


# speedup vs baseline: 1.0229x; 1.0229x over previous
import functools

import jax
import jax.numpy as jnp
from jax import lax
from jax.experimental import pallas as pl
from jax.experimental.pallas import tpu as pltpu
from jax.experimental.pallas import tpu_sc as plsc

N = 10000
D = 128
K = 64
E = 320000
TEMP = 0.5
LAMDA = 0.01

CHUNK = 80
NCORES = 2
NSUB = 16
NWORK = NCORES * NSUB
NCH = (E // NWORK) // CHUNK
ZROWS = N // NSUB
SROWS = N // NSUB
DEGW = 16
NBUF = 5

RB = 1000
GRID = N // RB

_HI = lax.Precision.HIGHEST


def _dot(a, b, dims, precision=None):
    return lax.dot_general(a, b, (dims, ((), ())),
                           preferred_element_type=jnp.float32,
                           precision=precision)


_sc_mesh = plsc.VectorSubcoreMesh(core_axis_name="c", subcore_axis_name="s")


@functools.partial(
    pl.kernel,
    out_type=[jax.ShapeDtypeStruct((NCORES, N, K), jnp.float32),
              jax.ShapeDtypeStruct((NCORES, N, DEGW), jnp.float32)],
    mesh=_sc_mesh,
    compiler_params=pltpu.CompilerParams(use_tc_tiling_on_sc=False),
    scratch_types=[
        pltpu.VMEM((NCH, CHUNK), jnp.int32),
        pltpu.VMEM((NCH, CHUNK), jnp.int32),
        [pltpu.VMEM((CHUNK, K), jnp.float32) for _ in range(NBUF)],
        pltpu.VMEM((CHUNK, DEGW), jnp.float32),
        pltpu.VMEM_SHARED((N, K), jnp.float32),
        pltpu.VMEM_SHARED((N, DEGW), jnp.float32),
        [pltpu.SemaphoreType.DMA for _ in range(NBUF)],
        [pltpu.SemaphoreType.DMA for _ in range(NBUF)],
        pltpu.SemaphoreType.DMA,
    ],
)
def _sc_apply_deg(hp, ei, zf, zd, ones, outf, outd,
                  srcv, dstv, rows, onesv, featS, degS, gsem, ssem, dsem):
    c = lax.axis_index("c")
    s = lax.axis_index("s")
    wid = c * NSUB + s
    row0 = s * SROWS
    pltpu.sync_copy(zf, featS.at[pl.ds(s * ZROWS, ZROWS)])
    pltpu.sync_copy(zd, degS.at[pl.ds(s * ZROWS, ZROWS)])
    pltpu.sync_copy(ei.at[0, wid], srcv)
    pltpu.sync_copy(ei.at[1, wid], dstv)
    pltpu.sync_copy(ones, onesv)
    plsc.subcore_barrier()

    for b in range(NBUF):
        pltpu.async_copy(hp.at[srcv.at[b]], rows[b], gsem[b])

    def step(g, carry):
        j0 = NBUF * g
        for b in range(NBUF):
            pltpu.make_async_copy(hp.at[srcv.at[j0 + b]], rows[b], gsem[b]).wait()
            pltpu.async_copy(rows[b], featS.at[dstv.at[j0 + b]], ssem[b], add=True)
            pltpu.async_copy(onesv, degS.at[dstv.at[j0 + b]], dsem, add=True)
        for b in range(NBUF):
            pltpu.make_async_copy(rows[b], featS.at[dstv.at[j0 + b]], ssem[b]).wait()
            pltpu.make_async_copy(onesv, degS.at[dstv.at[j0 + b]], dsem).wait()

            @pl.when(j0 + NBUF + b < NCH)
            def _():
                pltpu.async_copy(hp.at[srcv.at[j0 + NBUF + b]], rows[b], gsem[b])
        return carry

    lax.fori_loop(0, NCH // NBUF, step, 0)
    plsc.subcore_barrier()
    pltpu.sync_copy(featS.at[pl.ds(row0, SROWS)],
                    outf.at[c, pl.ds(row0, SROWS)])
    pltpu.sync_copy(degS.at[pl.ds(row0, SROWS)],
                    outd.at[c, pl.ds(row0, SROWS)])


@functools.partial(
    pl.kernel,
    out_type=[jax.ShapeDtypeStruct((NCORES, N, K), jnp.float32)],
    mesh=_sc_mesh,
    compiler_params=pltpu.CompilerParams(use_tc_tiling_on_sc=False),
    scratch_types=[
        pltpu.VMEM((NCH, CHUNK), jnp.int32),
        pltpu.VMEM((NCH, CHUNK), jnp.int32),
        [pltpu.VMEM((CHUNK, K), jnp.float32) for _ in range(NBUF)],
        pltpu.VMEM_SHARED((N, K), jnp.float32),
        [pltpu.SemaphoreType.DMA for _ in range(NBUF)],
        [pltpu.SemaphoreType.DMA for _ in range(NBUF)],
    ],
)
def _sc_apply(hp, ei, zf, outf, srcv, dstv, rows, featS, gsem, ssem):
    c = lax.axis_index("c")
    s = lax.axis_index("s")
    wid = c * NSUB + s
    row0 = s * SROWS
    pltpu.sync_copy(zf, featS.at[pl.ds(s * ZROWS, ZROWS)])
    pltpu.sync_copy(ei.at[0, wid], srcv)
    pltpu.sync_copy(ei.at[1, wid], dstv)
    plsc.subcore_barrier()

    for b in range(NBUF):
        pltpu.async_copy(hp.at[srcv.at[b]], rows[b], gsem[b])

    def step(g, carry):
        j0 = NBUF * g
        for b in range(NBUF):
            pltpu.make_async_copy(hp.at[srcv.at[j0 + b]], rows[b], gsem[b]).wait()
            pltpu.async_copy(rows[b], featS.at[dstv.at[j0 + b]], ssem[b], add=True)
        for b in range(NBUF):
            pltpu.make_async_copy(rows[b], featS.at[dstv.at[j0 + b]], ssem[b]).wait()

            @pl.when(j0 + NBUF + b < NCH)
            def _():
                pltpu.async_copy(hp.at[srcv.at[j0 + NBUF + b]], rows[b], gsem[b])
        return carry

    lax.fori_loop(0, NCH // NBUF, step, 0)
    plsc.subcore_barrier()
    pltpu.sync_copy(featS.at[pl.ds(row0, SROWS)],
                    outf.at[c, pl.ds(row0, SROWS)])


def _mm1_body(x_ref, w_ref, b_ref, o_ref):
    o_ref[...] = _dot(x_ref[...], w_ref[...], ((1,), (0,))) + b_ref[...]


_mm1 = pl.pallas_call(
    _mm1_body,
    grid=(GRID,),
    in_specs=[pl.BlockSpec((RB, D), lambda i: (i, 0)),
              pl.BlockSpec((D, K), lambda i: (0, 0)),
              pl.BlockSpec((1, K), lambda i: (0, 0))],
    out_specs=pl.BlockSpec((RB, K), lambda i: (i, 0)),
    out_shape=jax.ShapeDtypeStruct((N, K), jnp.float32),
)


def _mid_body(f0_ref, f1_ref, d0_ref, d1_ref, w_ref, b_ref, o_ref):
    deg = jnp.maximum(d0_ref[0][:, 0:1] + d1_ref[0][:, 0:1], 1.0)
    h = jnp.maximum((f0_ref[0] + f1_ref[0]) / deg, 0.0)
    o_ref[...] = _dot(h, w_ref[...], ((1,), (0,))) + b_ref[...]


_mid = pl.pallas_call(
    _mid_body,
    grid=(GRID,),
    in_specs=[pl.BlockSpec((1, RB, K), lambda i: (0, i, 0)),
              pl.BlockSpec((1, RB, K), lambda i: (1, i, 0)),
              pl.BlockSpec((1, RB, DEGW), lambda i: (0, i, 0)),
              pl.BlockSpec((1, RB, DEGW), lambda i: (1, i, 0)),
              pl.BlockSpec((K, K), lambda i: (0, 0)),
              pl.BlockSpec((1, K), lambda i: (0, 0))],
    out_specs=pl.BlockSpec((RB, K), lambda i: (i, 0)),
    out_shape=jax.ShapeDtypeStruct((N, K), jnp.float32),
)


def _final_body(f0_ref, f1_ref, d0_ref, d1_ref, emb_ref, eye_ref,
                gamma_ref, cidx_ref, loss_ref, miu_ref,
                gbuf, gsum_s, miun_s, tsum_s, tmiun_s):
    i = pl.program_id(0)

    @pl.when(i < GRID)
    def _():
        deg = jnp.maximum(d0_ref[0][:, 0:1] + d1_ref[0][:, 0:1], 1.0)
        x = (f0_ref[0] + f1_ref[0]) / deg
        m = jnp.max(x, axis=-1, keepdims=True)
        e = jnp.exp(x - m)
        gamma = e / jnp.sum(e, axis=-1, keepdims=True)
        gamma_ref[...] = gamma
        gbuf[pl.ds(i * RB, RB), :] = gamma
        cols = lax.broadcasted_iota(jnp.int32, (RB, K), 1)
        cidx_ref[...] = jnp.min(jnp.where(x >= m, cols, K), axis=-1,
                                keepdims=True)
        gs = jnp.sum(gamma, axis=0, keepdims=True)
        mn = _dot(gamma, emb_ref[...], ((0,), (0,)))

        @pl.when(i == 0)
        def _():
            gsum_s[...] = gs
            miun_s[...] = mn

        @pl.when(i > 0)
        def _():
            gsum_s[...] += gs
            miun_s[...] += mn

    @pl.when((i >= GRID) & (i < 2 * GRID))
    def _():
        g = gbuf[pl.ds((i - GRID) * RB, RB), :]
        y = g * g / gsum_s[...]
        m = jnp.max(y, axis=-1, keepdims=True)
        e = jnp.exp(y - m)
        tg = e / jnp.sum(e, axis=-1, keepdims=True)
        ts = jnp.sum(tg, axis=0, keepdims=True)
        tm = _dot(tg, emb_ref[...], ((0,), (0,)))

        @pl.when(i == GRID)
        def _():
            tsum_s[...] = ts
            tmiun_s[...] = tm

        @pl.when(i > GRID)
        def _():
            tsum_s[...] += ts
            tmiun_s[...] += tm

    @pl.when(i == 2 * GRID)
    def _():
        gs = gsum_s[...]
        eye = eye_ref[...]
        gcol = _dot(eye, gs, ((1,), (1,)), precision=_HI)
        tcol = _dot(eye, tsum_s[...], ((1,), (1,)), precision=_HI)
        miu = miun_s[...] / gcol
        tmiu = tmiun_s[...] / tcol
        miu_ref[...] = miu
        na = jnp.maximum(jnp.sqrt(jnp.sum(miu * miu, axis=-1, keepdims=True)),
                         1e-8)
        nb = jnp.maximum(jnp.sqrt(jnp.sum(tmiu * tmiu, axis=-1, keepdims=True)),
                         1e-8)
        a = miu / na
        b = tmiu / nb
        sim = _dot(a, b, ((1,), (1,))) / TEMP
        rm = jnp.max(sim, axis=-1, keepdims=True)
        lse_r = jnp.log(jnp.sum(jnp.exp(sim - rm), axis=-1, keepdims=True)) + rm
        cm = jnp.max(sim, axis=0, keepdims=True)
        lse_c = jnp.log(jnp.sum(jnp.exp(sim - cm), axis=0, keepdims=True)) + cm
        diag_ab = jnp.sum((sim - lse_r) * eye) / K
        diag_ba = jnp.sum((sim - lse_c) * eye) / K
        cl = -0.5 * (diag_ab + diag_ba)
        reg = jnp.mean(gs * gs) * LAMDA
        loss_ref[...] = jnp.broadcast_to(cl + reg, (1, 1))


def _cap(i):
    return jnp.minimum(i, GRID - 1)


_final = pl.pallas_call(
    _final_body,
    grid=(2 * GRID + 1,),
    in_specs=[pl.BlockSpec((1, RB, K), lambda i: (0, _cap(i), 0)),
              pl.BlockSpec((1, RB, K), lambda i: (1, _cap(i), 0)),
              pl.BlockSpec((1, RB, DEGW), lambda i: (0, _cap(i), 0)),
              pl.BlockSpec((1, RB, DEGW), lambda i: (1, _cap(i), 0)),
              pl.BlockSpec((RB, D),
                           lambda i: (jnp.where(i < GRID, i,
                                                _cap(i - GRID)), 0)),
              pl.BlockSpec((K, K), lambda i: (0, 0))],
    out_specs=[pl.BlockSpec((RB, K), lambda i: (_cap(i), 0)),
               pl.BlockSpec((RB, 1), lambda i: (_cap(i), 0)),
               pl.BlockSpec((1, 1), lambda i: (0, 0)),
               pl.BlockSpec((K, D), lambda i: (0, 0))],
    out_shape=[jax.ShapeDtypeStruct((N, K), jnp.float32),
               jax.ShapeDtypeStruct((N, 1), jnp.int32),
               jax.ShapeDtypeStruct((1, 1), jnp.float32),
               jax.ShapeDtypeStruct((K, D), jnp.float32)],
    scratch_shapes=[pltpu.VMEM((N, K), jnp.float32),
                    pltpu.VMEM((1, K), jnp.float32),
                    pltpu.VMEM((K, D), jnp.float32),
                    pltpu.VMEM((1, K), jnp.float32),
                    pltpu.VMEM((K, D), jnp.float32)],
)


def kernel(embeds, edge_index, W0, b0, W1, b1):
    ei = edge_index.reshape(2, NWORK, NCH, CHUNK)
    zf = jnp.zeros((ZROWS, K), jnp.float32)
    zd = jnp.zeros((ZROWS, DEGW), jnp.float32)
    ones = jnp.ones((CHUNK, DEGW), jnp.float32)
    eye = jnp.eye(K, dtype=jnp.float32)

    hp1 = _mm1(embeds, W0, b0.reshape(1, K))
    aggf1, aggd = _sc_apply_deg(hp1, ei, zf, zd, ones)
    hp2 = _mid(aggf1, aggf1, aggd, aggd, W1, b1.reshape(1, K))
    (aggf2,) = _sc_apply(hp2, ei, zf)
    gamma, cidx, loss11, miu = _final(aggf2, aggf2, aggd, aggd, embeds, eye)
    return (loss11[0, 0], gamma, cidx.reshape(N), miu)


# speedup vs baseline: 1.0317x; 1.0086x over previous
import functools

import jax
import jax.numpy as jnp
from jax import lax
from jax.experimental import pallas as pl
from jax.experimental.pallas import tpu as pltpu
from jax.experimental.pallas import tpu_sc as plsc

N = 10000
D = 128
K = 64
E = 320000
TEMP = 0.5
LAMDA = 0.01

CHUNK = 80
NCORES = 2
NSUB = 16
NWORK = NCORES * NSUB
NCH = (E // NWORK) // CHUNK
ZROWS = N // NSUB
SROWS = N // NSUB
DEGW = 16
NBUF = 5

RB = 1000
GRID = N // RB

_HI = lax.Precision.HIGHEST


def _dot(a, b, dims, precision=None):
    return lax.dot_general(a, b, (dims, ((), ())),
                           preferred_element_type=jnp.float32,
                           precision=precision)


_sc_mesh = plsc.VectorSubcoreMesh(core_axis_name="c", subcore_axis_name="s")


@functools.partial(
    pl.kernel,
    out_type=[jax.ShapeDtypeStruct((NCORES, N, K), jnp.float32),
              jax.ShapeDtypeStruct((NCORES, N, DEGW), jnp.float32)],
    mesh=_sc_mesh,
    compiler_params=pltpu.CompilerParams(use_tc_tiling_on_sc=False),
    scratch_types=[
        pltpu.VMEM((NCH, CHUNK), jnp.int32),
        pltpu.VMEM((NCH, CHUNK), jnp.int32),
        [pltpu.VMEM((CHUNK, K), jnp.float32) for _ in range(NBUF)],
        pltpu.VMEM((CHUNK, DEGW), jnp.float32),
        pltpu.VMEM_SHARED((N, K), jnp.float32),
        pltpu.VMEM_SHARED((N, DEGW), jnp.float32),
        [pltpu.SemaphoreType.DMA for _ in range(NBUF)],
        [pltpu.SemaphoreType.DMA for _ in range(NBUF)],
        pltpu.SemaphoreType.DMA,
    ],
)
def _sc_apply_deg(hp, ei, zf, zd, ones, outf, outd,
                  srcv, dstv, rows, onesv, featS, degS, gsem, ssem, dsem):
    c = lax.axis_index("c")
    s = lax.axis_index("s")
    wid = c * NSUB + s
    row0 = s * SROWS
    pltpu.sync_copy(zf, featS.at[pl.ds(s * ZROWS, ZROWS)])
    pltpu.sync_copy(zd, degS.at[pl.ds(s * ZROWS, ZROWS)])
    pltpu.sync_copy(ei.at[0, wid], srcv)
    pltpu.sync_copy(ei.at[1, wid], dstv)
    pltpu.sync_copy(ones, onesv)
    plsc.subcore_barrier()

    for b in range(NBUF):
        pltpu.async_copy(hp.at[srcv.at[b]], rows[b], gsem[b])

    def step(g, carry):
        j0 = NBUF * g
        for b in range(NBUF):
            pltpu.make_async_copy(hp.at[srcv.at[j0 + b]], rows[b], gsem[b]).wait()
            pltpu.async_copy(rows[b], featS.at[dstv.at[j0 + b]], ssem[b], add=True)
            pltpu.async_copy(onesv, degS.at[dstv.at[j0 + b]], dsem, add=True)
        for b in range(NBUF):
            pltpu.make_async_copy(rows[b], featS.at[dstv.at[j0 + b]], ssem[b]).wait()
            pltpu.make_async_copy(onesv, degS.at[dstv.at[j0 + b]], dsem).wait()

            @pl.when(j0 + NBUF + b < NCH)
            def _():
                pltpu.async_copy(hp.at[srcv.at[j0 + NBUF + b]], rows[b], gsem[b])
        return carry

    lax.fori_loop(0, NCH // NBUF, step, 0)
    plsc.subcore_barrier()
    pltpu.sync_copy(featS.at[pl.ds(row0, SROWS)],
                    outf.at[c, pl.ds(row0, SROWS)])
    pltpu.sync_copy(degS.at[pl.ds(row0, SROWS)],
                    outd.at[c, pl.ds(row0, SROWS)])


@functools.partial(
    pl.kernel,
    out_type=[jax.ShapeDtypeStruct((NCORES, N, K), jnp.float32)],
    mesh=_sc_mesh,
    compiler_params=pltpu.CompilerParams(use_tc_tiling_on_sc=False),
    scratch_types=[
        pltpu.VMEM((NCH, CHUNK), jnp.int32),
        pltpu.VMEM((NCH, CHUNK), jnp.int32),
        [pltpu.VMEM((CHUNK, K), jnp.float32) for _ in range(NBUF)],
        pltpu.VMEM_SHARED((N, K), jnp.float32),
        [pltpu.SemaphoreType.DMA for _ in range(NBUF)],
        [pltpu.SemaphoreType.DMA for _ in range(NBUF)],
    ],
)
def _sc_apply(hp, ei, zf, outf, srcv, dstv, rows, featS, gsem, ssem):
    c = lax.axis_index("c")
    s = lax.axis_index("s")
    wid = c * NSUB + s
    row0 = s * SROWS
    pltpu.sync_copy(zf, featS.at[pl.ds(s * ZROWS, ZROWS)])
    pltpu.sync_copy(ei.at[0, wid], srcv)
    pltpu.sync_copy(ei.at[1, wid], dstv)
    plsc.subcore_barrier()

    for b in range(NBUF):
        pltpu.async_copy(hp.at[srcv.at[b]], rows[b], gsem[b])

    def step(g, carry):
        j0 = NBUF * g
        for b in range(NBUF):
            pltpu.make_async_copy(hp.at[srcv.at[j0 + b]], rows[b], gsem[b]).wait()
            pltpu.async_copy(rows[b], featS.at[dstv.at[j0 + b]], ssem[b], add=True)
        for b in range(NBUF):
            pltpu.make_async_copy(rows[b], featS.at[dstv.at[j0 + b]], ssem[b]).wait()

            @pl.when(j0 + NBUF + b < NCH)
            def _():
                pltpu.async_copy(hp.at[srcv.at[j0 + NBUF + b]], rows[b], gsem[b])
        return carry

    lax.fori_loop(0, NCH // NBUF, step, 0)
    plsc.subcore_barrier()
    pltpu.sync_copy(featS.at[pl.ds(row0, SROWS)],
                    outf.at[c, pl.ds(row0, SROWS)])


def _mm1_body(x_ref, w_ref, b_ref, o_ref):
    o_ref[...] = _dot(x_ref[...], w_ref[...], ((1,), (0,))) + b_ref[...]


_mm1 = pl.pallas_call(
    _mm1_body,
    grid=(GRID,),
    in_specs=[pl.BlockSpec((RB, D), lambda i: (i, 0)),
              pl.BlockSpec((D, K), lambda i: (0, 0)),
              pl.BlockSpec((1, K), lambda i: (0, 0))],
    out_specs=pl.BlockSpec((RB, K), lambda i: (i, 0)),
    out_shape=jax.ShapeDtypeStruct((N, K), jnp.float32),
)


def _mid_body(f0_ref, f1_ref, d0_ref, d1_ref, w_ref, b_ref, o_ref):
    deg = jnp.maximum(d0_ref[0][:, 0:1] + d1_ref[0][:, 0:1], 1.0)
    h = jnp.maximum((f0_ref[0] + f1_ref[0]) / deg, 0.0)
    o_ref[...] = _dot(h, w_ref[...], ((1,), (0,))) + b_ref[...]


_mid = pl.pallas_call(
    _mid_body,
    grid=(GRID,),
    in_specs=[pl.BlockSpec((1, RB, K), lambda i: (0, i, 0)),
              pl.BlockSpec((1, RB, K), lambda i: (1, i, 0)),
              pl.BlockSpec((1, RB, DEGW), lambda i: (0, i, 0)),
              pl.BlockSpec((1, RB, DEGW), lambda i: (1, i, 0)),
              pl.BlockSpec((K, K), lambda i: (0, 0)),
              pl.BlockSpec((1, K), lambda i: (0, 0))],
    out_specs=pl.BlockSpec((RB, K), lambda i: (i, 0)),
    out_shape=jax.ShapeDtypeStruct((N, K), jnp.float32),
)


def _final_body(f0_ref, f1_ref, d0_ref, d1_ref, emb_ref, eye_ref,
                gamma_ref, cidx_ref, loss_ref, miu_ref,
                gbuf, ebuf, gsum_s, miun_s, tsum_s, tmiun_s):
    i = pl.program_id(0)

    @pl.when(i < GRID)
    def _():
        deg = jnp.maximum(d0_ref[0][:, 0:1] + d1_ref[0][:, 0:1], 1.0)
        x = (f0_ref[0] + f1_ref[0]) / deg
        m = jnp.max(x, axis=-1, keepdims=True)
        e = jnp.exp(x - m)
        gamma = e / jnp.sum(e, axis=-1, keepdims=True)
        gamma_ref[...] = gamma
        gbuf[pl.ds(i * RB, RB), :] = gamma
        ebuf[pl.ds(i * RB, RB), :] = emb_ref[...]
        cols = lax.broadcasted_iota(jnp.int32, (RB, K), 1)
        cidx_ref[...] = jnp.min(jnp.where(x >= m, cols, K), axis=-1,
                                keepdims=True)
        gs = jnp.sum(gamma, axis=0, keepdims=True)
        mn = _dot(gamma, emb_ref[...], ((0,), (0,)))

        @pl.when(i == 0)
        def _():
            gsum_s[...] = gs
            miun_s[...] = mn

        @pl.when(i > 0)
        def _():
            gsum_s[...] += gs
            miun_s[...] += mn

    @pl.when((i >= GRID) & (i < 2 * GRID))
    def _():
        g = gbuf[pl.ds((i - GRID) * RB, RB), :]
        em = ebuf[pl.ds((i - GRID) * RB, RB), :]
        y = g * g / gsum_s[...]
        m = jnp.max(y, axis=-1, keepdims=True)
        e = jnp.exp(y - m)
        tg = e / jnp.sum(e, axis=-1, keepdims=True)
        ts = jnp.sum(tg, axis=0, keepdims=True)
        tm = _dot(tg, em, ((0,), (0,)))

        @pl.when(i == GRID)
        def _():
            tsum_s[...] = ts
            tmiun_s[...] = tm

        @pl.when(i > GRID)
        def _():
            tsum_s[...] += ts
            tmiun_s[...] += tm

    @pl.when(i == 2 * GRID)
    def _():
        gs = gsum_s[...]
        eye = eye_ref[...]
        gcol = _dot(eye, gs, ((1,), (1,)), precision=_HI)
        tcol = _dot(eye, tsum_s[...], ((1,), (1,)), precision=_HI)
        miu = miun_s[...] / gcol
        tmiu = tmiun_s[...] / tcol
        miu_ref[...] = miu
        na = jnp.maximum(jnp.sqrt(jnp.sum(miu * miu, axis=-1, keepdims=True)),
                         1e-8)
        nb = jnp.maximum(jnp.sqrt(jnp.sum(tmiu * tmiu, axis=-1, keepdims=True)),
                         1e-8)
        a = miu / na
        b = tmiu / nb
        sim = _dot(a, b, ((1,), (1,))) / TEMP
        rm = jnp.max(sim, axis=-1, keepdims=True)
        lse_r = jnp.log(jnp.sum(jnp.exp(sim - rm), axis=-1, keepdims=True)) + rm
        cm = jnp.max(sim, axis=0, keepdims=True)
        lse_c = jnp.log(jnp.sum(jnp.exp(sim - cm), axis=0, keepdims=True)) + cm
        diag_ab = jnp.sum((sim - lse_r) * eye) / K
        diag_ba = jnp.sum((sim - lse_c) * eye) / K
        cl = -0.5 * (diag_ab + diag_ba)
        reg = jnp.mean(gs * gs) * LAMDA
        loss_ref[...] = jnp.broadcast_to(cl + reg, (1, 1))


def _cap(i):
    return jnp.minimum(i, GRID - 1)


_final = pl.pallas_call(
    _final_body,
    grid=(2 * GRID + 1,),
    in_specs=[pl.BlockSpec((1, RB, K), lambda i: (0, _cap(i), 0)),
              pl.BlockSpec((1, RB, K), lambda i: (1, _cap(i), 0)),
              pl.BlockSpec((1, RB, DEGW), lambda i: (0, _cap(i), 0)),
              pl.BlockSpec((1, RB, DEGW), lambda i: (1, _cap(i), 0)),
              pl.BlockSpec((RB, D), lambda i: (_cap(i), 0)),
              pl.BlockSpec((K, K), lambda i: (0, 0))],
    out_specs=[pl.BlockSpec((RB, K), lambda i: (_cap(i), 0)),
               pl.BlockSpec((RB, 1), lambda i: (_cap(i), 0)),
               pl.BlockSpec((1, 1), lambda i: (0, 0)),
               pl.BlockSpec((K, D), lambda i: (0, 0))],
    out_shape=[jax.ShapeDtypeStruct((N, K), jnp.float32),
               jax.ShapeDtypeStruct((N, 1), jnp.int32),
               jax.ShapeDtypeStruct((1, 1), jnp.float32),
               jax.ShapeDtypeStruct((K, D), jnp.float32)],
    scratch_shapes=[pltpu.VMEM((N, K), jnp.float32),
                    pltpu.VMEM((N, D), jnp.float32),
                    pltpu.VMEM((1, K), jnp.float32),
                    pltpu.VMEM((K, D), jnp.float32),
                    pltpu.VMEM((1, K), jnp.float32),
                    pltpu.VMEM((K, D), jnp.float32)],
)


def kernel(embeds, edge_index, W0, b0, W1, b1):
    ei = edge_index.reshape(2, NWORK, NCH, CHUNK)
    zf = jnp.zeros((ZROWS, K), jnp.float32)
    zd = jnp.zeros((ZROWS, DEGW), jnp.float32)
    ones = jnp.ones((CHUNK, DEGW), jnp.float32)
    eye = jnp.eye(K, dtype=jnp.float32)

    hp1 = _mm1(embeds, W0, b0.reshape(1, K))
    aggf1, aggd = _sc_apply_deg(hp1, ei, zf, zd, ones)
    hp2 = _mid(aggf1, aggf1, aggd, aggd, W1, b1.reshape(1, K))
    (aggf2,) = _sc_apply(hp2, ei, zf)
    gamma, cidx, loss11, miu = _final(aggf2, aggf2, aggd, aggd, embeds, eye)
    return (loss11[0, 0], gamma, cidx.reshape(N), miu)
